# baseline (speedup 1.0000x reference)
import jax
import jax.numpy as jnp
from jax import lax
from jax.experimental import pallas as pl
from jax.experimental.pallas import tpu as pltpu
from jax.experimental.pallas import tpu_sc as plsc

_BATCH = 16384
_NCOLS = 2048
_NGROUPS = 32
_GSIZE = 4
_USED = _NGROUPS * _GSIZE

_INFO = plsc.get_sparse_core_info()
_NC = _INFO.num_cores
_NS = _INFO.num_subcores
_LANES = _INFO.num_lanes
_NW = _NC * _NS

_SC_ROWS = 4096
_TC_ROWS = _BATCH - _SC_ROWS
_ROWS_PER_W = _SC_ROWS // _NW
_NCHUNK = 4
_CH = _ROWS_PER_W // _NCHUNK

_NSTREAM = 8
_TC_BLK = 512
_OBLK = _NSTREAM * _TC_BLK
_TC_GRID = _TC_ROWS // _OBLK


def _sc_body(x_hbm, out_hbm, xbuf, obuf, in_sems, out_sems):
    wid = lax.axis_index("s") * _NC + lax.axis_index("c")
    base = _TC_ROWS + wid * _ROWS_PER_W

    in_copies = [
        pltpu.async_copy(
            x_hbm.at[pl.ds(base + k * _CH, _CH), pl.ds(0, _USED)],
            xbuf.at[pl.ds(k * _CH, _CH)],
            in_sems.at[k],
        )
        for k in range(_NCHUNK)
    ]

    lane = lax.iota(jnp.int32, _LANES)
    cols = [
        [lane * _GSIZE + (b * _LANES * _GSIZE + j) for j in range(_GSIZE)]
        for b in range(_NGROUPS // _LANES)
    ]
    scale = jnp.float32(1.0 / _GSIZE)

    def row_step(r, carry):
        row = xbuf.at[r]
        for b in range(_NGROUPS // _LANES):
            acc = plsc.load_gather(row, [cols[b][0]])
            for j in range(1, _GSIZE):
                acc = acc + plsc.load_gather(row, [cols[b][j]])
            obuf[r, pl.ds(b * _LANES, _LANES)] = acc * scale
        return carry

    out_copies = []
    for k in range(_NCHUNK):
        in_copies[k].wait()
        lax.fori_loop(k * _CH, (k + 1) * _CH, row_step, 0, unroll=4)
        out_copies.append(
            pltpu.async_copy(
                obuf.at[pl.ds(k * _CH, _CH)],
                out_hbm.at[pl.ds(wid * _ROWS_PER_W + k * _CH, _CH)],
                out_sems.at[k],
            )
        )
    for c in out_copies:
        c.wait()


def _sc_pool(x):
    mesh = plsc.VectorSubcoreMesh(core_axis_name="c", subcore_axis_name="s")
    return pl.kernel(
        _sc_body,
        out_type=jax.ShapeDtypeStruct((_SC_ROWS, _NGROUPS), jnp.float32),
        mesh=mesh,
        compiler_params=pltpu.CompilerParams(needs_layout_passes=False),
        scratch_types=[
            pltpu.VMEM((_ROWS_PER_W, _USED), jnp.float32),
            pltpu.VMEM((_ROWS_PER_W, _NGROUPS), jnp.float32),
            pltpu.SemaphoreType.DMA((_NCHUNK,)),
            pltpu.SemaphoreType.DMA((_NCHUNK,)),
        ],
    )(x)


def _tc_body(*refs):
    x_refs, o_ref = refs[:_NSTREAM], refs[_NSTREAM]
    k = lax.broadcasted_iota(jnp.int32, (_USED, _NGROUPS), 0)
    i = lax.broadcasted_iota(jnp.int32, (_USED, _NGROUPS), 1)
    w = jnp.where(k // _GSIZE == i, jnp.float32(1.0 / _GSIZE), jnp.float32(0.0))
    for q, x_ref in enumerate(x_refs):
        o_ref[q * _TC_BLK:(q + 1) * _TC_BLK, :] = jnp.dot(
            x_ref[...], w, preferred_element_type=jnp.float32,
            precision=lax.Precision.HIGHEST)


def _tc_pool(x):
    def in_map(q):
        return lambda i: (i * _NSTREAM + q, 0)

    return pl.pallas_call(
        _tc_body,
        grid=(_TC_GRID,),
        in_specs=[pl.BlockSpec((_TC_BLK, _USED), in_map(q))
                  for q in range(_NSTREAM)],
        out_specs=pl.BlockSpec((_OBLK, _NGROUPS), lambda i: (i, 0)),
        out_shape=jax.ShapeDtypeStruct((_TC_ROWS, _NGROUPS), jnp.float32),
    )(*([x] * _NSTREAM))


@jax.jit
def _pooled_mean(x):
    out_sc = _sc_pool(x)
    out_tc = _tc_pool(x)
    out = jnp.zeros((_BATCH, _NGROUPS), jnp.float32)
    out = jax.lax.dynamic_update_slice(out, out_tc, (0, 0))
    return jax.lax.dynamic_update_slice(out, out_sc, (_TC_ROWS, 0))


def kernel(gene_set_features):
    return _pooled_mean(gene_set_features)


# speedup vs baseline: 3.4242x; 3.4242x over previous
import jax
import jax.numpy as jnp
from jax import lax
from jax.experimental import pallas as pl

_BATCH = 16384
_NGROUPS = 32
_GSIZE = 4
_USED = _NGROUPS * _GSIZE

_NSTREAM = 8
_TC_BLK = 512
_OBLK = _NSTREAM * _TC_BLK
_GRID = _BATCH // _OBLK


def _tc_body(*refs):
    x_refs, o_ref = refs[:_NSTREAM], refs[_NSTREAM]
    k = lax.broadcasted_iota(jnp.int32, (_USED, _NGROUPS), 0)
    i = lax.broadcasted_iota(jnp.int32, (_USED, _NGROUPS), 1)
    w = jnp.where(k // _GSIZE == i, jnp.float32(1.0 / _GSIZE), jnp.float32(0.0))
    for q, x_ref in enumerate(x_refs):
        o_ref[:, q * _TC_BLK:(q + 1) * _TC_BLK] = lax.dot_general(
            w, x_ref[...],
            dimension_numbers=(((0,), (1,)), ((), ())),
            preferred_element_type=jnp.float32,
            precision=lax.Precision.HIGHEST)


@jax.jit
def _pooled_mean(x):
    def in_map(q):
        return lambda i: (i * _NSTREAM + q, 0)

    out_t = pl.pallas_call(
        _tc_body,
        grid=(_GRID,),
        in_specs=[pl.BlockSpec((_TC_BLK, _USED), in_map(q))
                  for q in range(_NSTREAM)],
        out_specs=pl.BlockSpec((_NGROUPS, _OBLK), lambda i: (0, i)),
        out_shape=jax.ShapeDtypeStruct((_NGROUPS, _BATCH), jnp.float32),
    )(*([x] * _NSTREAM))
    return jnp.transpose(out_t)


def kernel(gene_set_features):
    return _pooled_mean(gene_set_features)
